# Initial kernel scaffold; baseline (speedup 1.0000x reference)
import jax
import jax.numpy as jnp
from jax.experimental import pallas as pl


def kernel(x, edge_index, edge_type, batch, W1, W1_root, b1, W2, W2_root, b2, fc1_W, fc1_b, bn_g, bn_b, bn_mean, bn_var, fc2_W, fc2_b, dec1_W, dec1_b, dec2_W, dec2_b):
    raise NotImplementedError("write your pallas kernel here")


# speedup vs baseline: 4.6728x; 4.6728x over previous
import functools

import jax
import jax.numpy as jnp
from jax import lax
from jax.experimental import pallas as pl
from jax.experimental.pallas import tpu as pltpu
from jax.experimental.pallas import tpu_sc as plsc

_N = 10000
_E = 320000
_R = 8
_NG = 64
_CHUNK = 80


def _sc_dims():
    info = plsc.get_sparse_core_info()
    nc, ns = info.num_cores, info.num_subcores
    return nc, ns, nc * ns


def _sc_gather(table, idx):
    nc, ns, nw = _sc_dims()
    bpw = _E // nw
    nchunk = bpw // _CHUNK
    w = table.shape[1]
    mesh = plsc.VectorSubcoreMesh(core_axis_name="c", subcore_axis_name="s")

    @functools.partial(
        pl.kernel,
        mesh=mesh,
        out_type=jax.ShapeDtypeStruct((_E, w), jnp.float32),
        scratch_types=[
            pltpu.VMEM((_CHUNK,), jnp.int32),
            pltpu.VMEM((_CHUNK, w), jnp.float32),
            pltpu.SemaphoreType.DMA,
        ],
    )
    def k(table_hbm, idx_hbm, out_hbm, idx_v, rows_v, sem):
        wid = lax.axis_index("s") * nc + lax.axis_index("c")
        for c in range(nchunk):
            base = wid * bpw + c * _CHUNK
            pltpu.sync_copy(idx_hbm.at[pl.ds(base, _CHUNK)], idx_v)
            pltpu.async_copy(table_hbm.at[idx_v], rows_v, sem).wait()
            pltpu.sync_copy(rows_v, out_hbm.at[pl.ds(base, _CHUNK)])

    return k(table, idx)


def _sc_scatter_add(vals, idx, nrows):
    nc, ns, nw = _sc_dims()
    bpw = _E // nw
    nchunk = bpw // _CHUNK
    idx_list = idx if isinstance(idx, (list, tuple)) else [idx]
    pw = vals.shape[1]
    npass = len(idx_list)
    half = nrows // npass
    acc_rows = half + 8
    rpt = 1000
    ndump = half // rpt
    zeros = jnp.zeros((acc_rows, pw), jnp.float32)
    mesh = plsc.VectorSubcoreMesh(core_axis_name="c", subcore_axis_name="s")

    @functools.partial(
        pl.kernel,
        mesh=mesh,
        out_type=jax.ShapeDtypeStruct((nc, npass, half, pw), jnp.float32),
        scratch_types=[
            pltpu.VMEM((1, _CHUNK), jnp.int32),
            pltpu.VMEM((_CHUNK, pw), jnp.float32),
            pltpu.VMEM_SHARED((acc_rows, pw), jnp.float32),
        ],
    )
    def k(*refs):
        vals_hbm = refs[0]
        idx_hbms = refs[1:1 + npass]
        zeros_hbm, out_hbm, idx_v, rows_v, acc_sh = refs[1 + npass:]
        cid = lax.axis_index("c")
        sid = lax.axis_index("s")
        wid = sid * nc + cid
        for p in range(npass):
            @pl.when(sid == 0)
            def _():
                pltpu.sync_copy(zeros_hbm, acc_sh)

            plsc.subcore_barrier()
            for c in range(nchunk):
                base = wid * bpw + c * _CHUNK
                pltpu.sync_copy(idx_hbms[p].at[pl.ds(base, _CHUNK)], idx_v.at[0])
                pltpu.sync_copy(vals_hbm.at[pl.ds(base, _CHUNK)], rows_v)
                pltpu.sync_copy(rows_v, acc_sh.at[idx_v.at[0]], add=True)
            plsc.subcore_barrier()

            @pl.when(sid < ndump)
            def _():
                r0 = sid * rpt
                pltpu.sync_copy(acc_sh.at[pl.ds(r0, rpt)],
                                out_hbm.at[cid, p, pl.ds(r0, rpt)])

            plsc.subcore_barrier()

    return k(vals, *idx_list, zeros)


def _tc_relation_transform(x, W):
    r, d, o = W.shape
    n = x.shape[0]

    def kern(x_ref, w_ref, o_ref):
        o_ref[0] = jnp.dot(x_ref[...], w_ref[0], preferred_element_type=jnp.float32)

    return pl.pallas_call(
        kern,
        grid=(r,),
        in_specs=[
            pl.BlockSpec((n, d), lambda i: (0, 0)),
            pl.BlockSpec((1, d, o), lambda i: (i, 0, 0)),
        ],
        out_specs=pl.BlockSpec((1, n, o), lambda i: (i, 0, 0)),
        out_shape=jax.ShapeDtypeStruct((r, n, o), jnp.float32),
    )(x, W)


def _tc_sum_partials(p):
    _, t, w = p.shape

    def kern(p_ref, o_ref):
        o_ref[...] = p_ref[0] + p_ref[1]

    return pl.pallas_call(
        kern,
        in_specs=[pl.BlockSpec((2, t, w), lambda: (0, 0, 0))],
        out_specs=pl.BlockSpec((t, w), lambda: (0, 0)),
        out_shape=jax.ShapeDtypeStruct((t, w), jnp.float32),
    )(p)


def _tc_scale(msg, cnt_rows, rel_oh):
    e = msg.shape[0]
    b = 4000

    def kern(m_ref, c_ref, r_ref, o_ref):
        cnt = jnp.sum(c_ref[:, :_R] * r_ref[...], axis=1, keepdims=True)
        o_ref[...] = m_ref[...] * (1.0 / jnp.maximum(cnt, 1.0))

    return pl.pallas_call(
        kern,
        grid=(e // b,),
        in_specs=[
            pl.BlockSpec((b, 128), lambda i: (i, 0)),
            pl.BlockSpec((b, 128), lambda i: (i, 0)),
            pl.BlockSpec((b, _R), lambda i: (i, 0)),
        ],
        out_specs=pl.BlockSpec((b, 128), lambda i: (i, 0)),
        out_shape=jax.ShapeDtypeStruct((e, 128), jnp.float32),
    )(msg, cnt_rows, rel_oh)


def _tc_combine(agg_p, x, W_root, b):
    n, d = x.shape
    o = W_root.shape[1]

    def kern(a_ref, x_ref, w_ref, b_ref, o_ref):
        root = jnp.dot(x_ref[...], w_ref[...], preferred_element_type=jnp.float32)
        agg = (a_ref[0] + a_ref[1])[:, :o]
        o_ref[...] = jax.nn.relu(agg + root + b_ref[...])

    return pl.pallas_call(
        kern,
        in_specs=[
            pl.BlockSpec((2, n, 128), lambda: (0, 0, 0)),
            pl.BlockSpec((n, d), lambda: (0, 0)),
            pl.BlockSpec((d, o), lambda: (0, 0)),
            pl.BlockSpec((1, o), lambda: (0, 0)),
        ],
        out_specs=pl.BlockSpec((n, o), lambda: (0, 0)),
        out_shape=jax.ShapeDtypeStruct((n, o), jnp.float32),
    )(agg_p, x, W_root, b)


def _tc_pool_mlp(h, onehot, fc1_W, fc1_b, bn_scale, bn_shift, fc2_W, fc2_b,
                 dec1_W, dec1_b, dec2_W, dec2_b):
    n, d = h.shape

    def kern(h_ref, oh_ref, fc1w, fc1b, bns, bnb, fc2w, fc2b, d1w, d1b, d2w,
             d2b, pred_ref, emb_ref, mx_ref):
        hv = h_ref[...]
        oh = oh_ref[...]
        cnt = jnp.sum(oh, axis=0)[:, None]
        total = jnp.dot(oh.T, hv, preferred_element_type=jnp.float32)
        mean = total / jnp.maximum(cnt, 1.0)

        for g in range(_NG):
            m = oh[:, g:g + 1]
            mxg = jnp.max(jnp.where(m > 0.0, hv, -1e30), axis=0, keepdims=True)
            mx_ref[g:g + 1, :] = mxg
        mx = jnp.where(cnt > 0.0, mx_ref[...], 0.0)
        g = mean + mx

        def leaky(v):
            return jnp.where(v >= 0.0, v, 0.2 * v)

        e = jnp.dot(g, fc1w[...], preferred_element_type=jnp.float32) + fc1b[...]
        e = leaky(e * bns[...] + bnb[...])
        emb = jnp.dot(e, fc2w[...], preferred_element_type=jnp.float32) + fc2b[...]
        dd = leaky(jnp.dot(emb, d1w[...], preferred_element_type=jnp.float32) + d1b[...])
        pred_ref[...] = jnp.dot(dd, d2w[...], preferred_element_type=jnp.float32) + d2b[...]
        emb_ref[...] = emb

    full = lambda *s: pl.BlockSpec(s, lambda: (0,) * len(s))
    return pl.pallas_call(
        kern,
        in_specs=[
            full(n, d), full(n, _NG),
            full(128, 64), full(1, 64), full(1, 64), full(1, 64),
            full(64, 16), full(1, 16),
            full(16, 64), full(1, 64), full(64, 32), full(1, 32),
        ],
        out_specs=[full(_NG, 32), full(_NG, 16)],
        out_shape=[
            jax.ShapeDtypeStruct((_NG, 32), jnp.float32),
            jax.ShapeDtypeStruct((_NG, 16), jnp.float32),
        ],
        scratch_shapes=[pltpu.VMEM((_NG, d), jnp.float32)],
    )(h, onehot, fc1_W, fc1_b, bn_scale, bn_shift, fc2_W, fc2_b,
      dec1_W, dec1_b, dec2_W, dec2_b)


def _rgcn_layer(h, comb_src, dst_passes, cnt_rows_e, rel_oh, W, W_root, b):
    r, d, o = W.shape
    W_pad = jnp.pad(W, ((0, 0), (0, 0), (0, 128 - o)))
    xw = _tc_relation_transform(h, W_pad)
    msg = _sc_gather(xw.reshape(r * _N, 128), comb_src)
    msg = _tc_scale(msg, cnt_rows_e, rel_oh)
    agg_p = _sc_scatter_add(msg, dst_passes, _N)
    agg_p = jnp.concatenate([agg_p[:, 0], agg_p[:, 1]], axis=1)
    return _tc_combine(agg_p, h, W_root, b.reshape(1, o))


def kernel(x, edge_index, edge_type, batch, W1, W1_root, b1, W2, W2_root, b2,
           fc1_W, fc1_b, bn_g, bn_b, bn_mean, bn_var, fc2_W, fc2_b,
           dec1_W, dec1_b, dec2_W, dec2_b):
    src, dst = edge_index[0], edge_index[1]
    comb_src = edge_type * _N + src

    rel_oh = (edge_type[:, None] == jnp.arange(_R, dtype=jnp.int32)[None, :]
              ).astype(jnp.float32)
    rel_oh128 = jnp.pad(rel_oh, ((0, 0), (0, 128 - _R)))
    half = _N // 2
    dst_passes = [jnp.where(dst < half, dst, half),
                  jnp.where(dst >= half, dst - half, half)]
    cnt_p = _sc_scatter_add(rel_oh128, dst_passes, _N)
    cnt = _tc_sum_partials(
        jnp.concatenate([cnt_p[:, 0], cnt_p[:, 1]], axis=1))
    cnt_rows_e = _sc_gather(cnt, dst)

    h = _rgcn_layer(x, comb_src, dst_passes, cnt_rows_e, rel_oh, W1, W1_root, b1)
    h = _rgcn_layer(h, comb_src, dst_passes, cnt_rows_e, rel_oh, W2, W2_root, b2)

    onehot = (batch[:, None] == jnp.arange(_NG, dtype=jnp.int32)[None, :]
              ).astype(jnp.float32)
    inv_std = 1.0 / jnp.sqrt(bn_var + 1e-5)
    bn_scale = (bn_g * inv_std).reshape(1, 64)
    bn_shift = (bn_b - bn_mean * bn_g * inv_std).reshape(1, 64)
    pred, emb = _tc_pool_mlp(
        h, onehot, fc1_W, fc1_b.reshape(1, 64), bn_scale, bn_shift,
        fc2_W, fc2_b.reshape(1, 16), dec1_W, dec1_b.reshape(1, 64),
        dec2_W, dec2_b.reshape(1, 32))
    return (pred, emb)
